# baseline (device time: 13463 ns/iter reference)
import jax
import jax.numpy as jnp
from jax import lax
from jax.experimental import pallas as pl
from jax.experimental.pallas import tpu as pltpu

CH_ROWS = 128


def kernel(x, dy, gamma):
    m, d = x.shape
    half = m // 2
    n_chunks = half // CH_ROWS

    def body(x_hbm, dy_hbm, out_ref, xv, dyv, acc_ref, comm_ref, pad_ref,
             x_sems, dy_sems, send_sems, recv_sems, out_sem):
        my_x = lax.axis_index("x")
        my_y = lax.axis_index("y")
        peers = [(my_x, 1 - my_y), (1 - my_x, my_y), (1 - my_x, 1 - my_y)]

        barrier = pltpu.get_barrier_semaphore()
        for p in peers:
            pl.semaphore_signal(
                barrier, inc=1, device_id=p,
                device_id_type=pl.DeviceIdType.MESH,
            )

        pad_ref[0, 0:8, 0:128] = jnp.zeros((8, 128), jnp.float32)

        row0 = my_x * half

        def make_copies(c):
            rows = pl.ds(row0 + c * CH_ROWS, CH_ROWS)
            cx = pltpu.make_async_copy(x_hbm.at[rows, :], xv.at[c],
                                       x_sems.at[c])
            cd = pltpu.make_async_copy(dy_hbm.at[rows, :], dyv.at[c],
                                       dy_sems.at[c])
            return cx, cd

        WINDOW = 4
        copies = [make_copies(c) for c in range(n_chunks)]
        for c in range(min(WINDOW, n_chunks)):
            copies[c][0].start()
            copies[c][1].start()

        dg = jnp.zeros((1, d), jnp.float32)
        db = jnp.zeros((1, d), jnp.float32)
        for c in range(n_chunks):
            cx, cd = copies[c]
            cx.wait()
            cd.wait()
            if c + WINDOW < n_chunks:
                copies[c + WINDOW][0].start()
                copies[c + WINDOW][1].start()
            xb = xv[c]
            dyb = dyv[c]
            mu = jnp.mean(xb, axis=1, keepdims=True)
            ex2 = jnp.mean(xb * xb, axis=1, keepdims=True)
            rstd = lax.rsqrt(ex2 - mu * mu + 1e-5)
            dg += jnp.sum(dyb * (xb * rstd - mu * rstd), axis=0,
                          keepdims=True)
            db += jnp.sum(dyb, axis=0, keepdims=True)
        acc_ref[...] = jnp.concatenate([dg, db], axis=0)

        pl.semaphore_wait(barrier, 3)
        rdmas = []
        for slot, p in enumerate(peers):
            r = pltpu.make_async_remote_copy(
                src_ref=acc_ref,
                dst_ref=comm_ref.at[slot],
                send_sem=send_sems.at[slot],
                recv_sem=recv_sems.at[slot],
                device_id=p,
                device_id_type=pl.DeviceIdType.MESH,
            )
            r.start()
            rdmas.append(r)
        for r in rdmas:
            r.wait()
        acc_ref[...] += comm_ref[0] + comm_ref[1] + comm_ref[2]
        out_copy = pltpu.make_async_copy(acc_ref, out_ref, out_sem)
        out_copy.start()
        out_copy.wait()

    return pl.pallas_call(
        body,
        out_shape=jax.ShapeDtypeStruct((2, d), jnp.float32),
        in_specs=[
            pl.BlockSpec(memory_space=pltpu.MemorySpace.HBM),
            pl.BlockSpec(memory_space=pltpu.MemorySpace.HBM),
        ],
        out_specs=pl.BlockSpec(memory_space=pltpu.MemorySpace.HBM),
        scratch_shapes=[
            pltpu.VMEM((n_chunks, CH_ROWS, d), jnp.float32),
            pltpu.VMEM((n_chunks, CH_ROWS, d), jnp.float32),
            pltpu.VMEM((2, d), jnp.float32),
            pltpu.VMEM((3, 2, d), jnp.float32),
            pltpu.VMEM((10, 1024, 1024), jnp.float32),
            pltpu.SemaphoreType.DMA((n_chunks,)),
            pltpu.SemaphoreType.DMA((n_chunks,)),
            pltpu.SemaphoreType.DMA((3,)),
            pltpu.SemaphoreType.DMA((3,)),
            pltpu.SemaphoreType.DMA,
        ],
        compiler_params=pltpu.CompilerParams(
            collective_id=0, vmem_limit_bytes=63 * 1024 * 1024
        ),
    )(x, dy)
